# baseline (speedup 1.0000x reference)
import functools

import jax
import jax.numpy as jnp
from jax import lax
from jax.experimental import pallas as pl
from jax.experimental.pallas import tpu as pltpu
from jax.experimental.pallas import tpu_sc as plsc

N_NODES = 10000
N_EDGES = 320000
CH = 128

NC = 2
NS = 16
NW = NC * NS
K = 120
CHUNKS = 85
EPW = CHUNKS * K
E_PAD = NW * EPW
NBUF = 2
N_PAD = 10240
RPT = N_PAD // NS


def _sc_aggregate(x, row1, col1, zeros):
    mesh = plsc.VectorSubcoreMesh(core_axis_name="c", subcore_axis_name="s")

    @functools.partial(
        pl.kernel,
        mesh=mesh,
        out_type=jax.ShapeDtypeStruct((NC, N_PAD, CH), jnp.float32),
        scratch_types=[
            pltpu.VMEM((NBUF, K), jnp.int32),
            pltpu.VMEM((NBUF, K), jnp.int32),
            pltpu.VMEM((K, CH), jnp.float32),
            pltpu.VMEM((K, CH), jnp.float32),
            pltpu.VMEM_SHARED((N_PAD, CH), jnp.float32),
            pltpu.SemaphoreType.DMA,
            pltpu.SemaphoreType.DMA,
            pltpu.SemaphoreType.DMA,
            pltpu.SemaphoreType.DMA,
        ],
    )
    def agg_kernel(x_hbm, row_hbm, col_hbm, z_hbm, out_hbm,
                   cbufs, rbufs, gbuf0, gbuf1, acc,
                   si0, si1, sg0, sg1):
        c = lax.axis_index("c")
        s = lax.axis_index("s")
        wid = c * NS + s
        base = wid * EPW

        gbuf = (gbuf0, gbuf1)
        sem_i = (si0, si1)
        sem_g = (sg0, sg1)

        def issue_idx(k, b):
            off = base + k * K
            pltpu.async_copy(col_hbm.at[pl.ds(off, K)], cbufs.at[b], sem_i[b])
            pltpu.async_copy(row_hbm.at[pl.ds(off, K)], rbufs.at[b], sem_i[b])

        def wait_idx(k, b):
            off = base + k * K
            pltpu.make_async_copy(col_hbm.at[pl.ds(off, K)], cbufs.at[b],
                                  sem_i[b]).wait()
            pltpu.make_async_copy(row_hbm.at[pl.ds(off, K)], rbufs.at[b],
                                  sem_i[b]).wait()

        def issue_gather(b):
            pltpu.async_copy(x_hbm.at[cbufs.at[b]], gbuf[b], sem_g[b])

        def wait_gather(b):
            pltpu.make_async_copy(x_hbm.at[cbufs.at[b]], gbuf[b],
                                  sem_g[b]).wait()

        issue_idx(0, 0)
        pltpu.sync_copy(z_hbm.at[pl.ds(s * RPT, RPT)],
                        acc.at[pl.ds(s * RPT, RPT)])
        wait_idx(0, 0)
        issue_gather(0)
        issue_idx(1, 1)
        plsc.subcore_barrier()

        def half(k, b):
            b2 = 1 - b
            wait_idx(k + 1, b2)
            issue_gather(b2)
            wait_gather(b)
            pltpu.sync_copy(gbuf[b], acc.at[rbufs.at[b]], add=True)
            issue_idx(k + 2, b)

        def body(g, carry):
            half(g * 2, 0)
            half(g * 2 + 1, 1)
            return carry

        lax.fori_loop(0, (CHUNKS - 1) // 2, body, 0)
        wait_gather(0)
        pltpu.sync_copy(gbuf[0], acc.at[rbufs.at[0]], add=True)
        wait_idx(CHUNKS, 1)

        plsc.subcore_barrier()
        pltpu.sync_copy(acc.at[pl.ds(s * RPT, RPT)],
                        out_hbm.at[c, pl.ds(s * RPT, RPT)])

    return agg_kernel(x, row1, col1, zeros)


ROWS_BLK = 2000


def _mm_body(p_ref, w_ref, o_ref):
    acc = p_ref[0] + p_ref[1]
    o_ref[...] = lax.dot_general(
        acc, w_ref[...], (((1,), (1,)), ((), ())),
        preferred_element_type=jnp.float32)


def _tc_matmul(partials, W_fc):
    return pl.pallas_call(
        _mm_body,
        grid=(N_NODES // ROWS_BLK,),
        in_specs=[
            pl.BlockSpec((NC, ROWS_BLK, CH), lambda i: (0, i, 0)),
            pl.BlockSpec((CH, CH), lambda i: (0, 0)),
        ],
        out_specs=pl.BlockSpec((ROWS_BLK, CH), lambda i: (i, 0)),
        out_shape=jax.ShapeDtypeStruct((N_NODES, CH), jnp.float32),
    )(partials, W_fc)


def kernel(x, edge_index, edge_attr, W_fc, W_edge, W_att):
    del edge_attr, W_edge, W_att
    ei = edge_index.astype(jnp.int32)
    pad = E_PAD - N_EDGES
    row1 = jnp.concatenate(
        [ei[0], jnp.full((pad,), N_NODES, jnp.int32),
         jnp.zeros((2 * K,), jnp.int32)])
    col1 = jnp.concatenate([ei[1], jnp.zeros((pad + 2 * K,), jnp.int32)])
    zeros = jnp.zeros((N_PAD, CH), jnp.float32)
    partials = _sc_aggregate(x, row1, col1, zeros)
    return _tc_matmul(partials[:, :N_NODES, :], W_fc)


# speedup vs baseline: 1.3100x; 1.3100x over previous
import functools

import jax
import jax.numpy as jnp
from jax import lax
from jax.experimental import pallas as pl
from jax.experimental.pallas import tpu as pltpu
from jax.experimental.pallas import tpu_sc as plsc

N_NODES = 10000
N_EDGES = 320000
CH = 128

NC = 2
NS = 16
NW = NC * NS
K = 88
CHUNKS = 115
EPW = CHUNKS * K
E_PAD = NW * EPW
NBUF = 2
N_PAD = 10240
RPT = N_PAD // NS


def _sc_aggregate(x, row1, col1, zeros):
    mesh = plsc.VectorSubcoreMesh(core_axis_name="c", subcore_axis_name="s")

    @functools.partial(
        pl.kernel,
        mesh=mesh,
        out_type=jax.ShapeDtypeStruct((NC, N_PAD, CH), jnp.float32),
        scratch_types=[
            pltpu.VMEM((NBUF, K), jnp.int32),
            pltpu.VMEM((NBUF, K), jnp.int32),
            pltpu.VMEM((K, CH), jnp.float32),
            pltpu.VMEM((K, CH), jnp.float32),
            pltpu.VMEM_SHARED((N_PAD, CH), jnp.float32),
            pltpu.SemaphoreType.DMA,
            pltpu.SemaphoreType.DMA,
            pltpu.SemaphoreType.DMA,
            pltpu.SemaphoreType.DMA,
        ],
    )
    def agg_kernel(x_hbm, row_hbm, col_hbm, z_hbm, out_hbm,
                   cbufs, rbufs, gbuf0, gbuf1, acc,
                   si0, si1, sg0, sg1):
        c = lax.axis_index("c")
        s = lax.axis_index("s")
        wid = c * NS + s
        base = wid * EPW

        gbuf = (gbuf0, gbuf1)
        sem_i = (si0, si1)
        sem_g = (sg0, sg1)

        def issue_idx(k, b):
            off = base + k * K
            pltpu.async_copy(col_hbm.at[pl.ds(off, K)], cbufs.at[b], sem_i[b])
            pltpu.async_copy(row_hbm.at[pl.ds(off, K)], rbufs.at[b], sem_i[b])

        def wait_idx(k, b):
            off = base + k * K
            pltpu.make_async_copy(col_hbm.at[pl.ds(off, K)], cbufs.at[b],
                                  sem_i[b]).wait()
            pltpu.make_async_copy(row_hbm.at[pl.ds(off, K)], rbufs.at[b],
                                  sem_i[b]).wait()

        def issue_gather(b):
            pltpu.async_copy(x_hbm.at[cbufs.at[b]], gbuf[b], sem_g[b])

        def wait_gather(b):
            pltpu.make_async_copy(x_hbm.at[cbufs.at[b]], gbuf[b],
                                  sem_g[b]).wait()

        issue_idx(0, 0)
        pltpu.sync_copy(z_hbm.at[pl.ds(s * RPT, RPT)],
                        acc.at[pl.ds(s * RPT, RPT)])
        wait_idx(0, 0)
        issue_gather(0)
        issue_idx(1, 1)
        plsc.subcore_barrier()

        def half(k, b):
            b2 = 1 - b
            wait_idx(k + 1, b2)
            issue_gather(b2)
            wait_gather(b)
            pltpu.sync_copy(gbuf[b], acc.at[rbufs.at[b]], add=True)
            issue_idx(k + 2, b)

        def body(g, carry):
            half(g * 2, 0)
            half(g * 2 + 1, 1)
            return carry

        lax.fori_loop(0, (CHUNKS - 1) // 2, body, 0)
        wait_gather(0)
        pltpu.sync_copy(gbuf[0], acc.at[rbufs.at[0]], add=True)
        wait_idx(CHUNKS, 1)

        plsc.subcore_barrier()
        pltpu.sync_copy(acc.at[pl.ds(s * RPT, RPT)],
                        out_hbm.at[c, pl.ds(s * RPT, RPT)])

    return agg_kernel(x, row1, col1, zeros)


ROWS_BLK = 2000


def _mm_body(p_ref, w_ref, o_ref):
    acc = p_ref[0] + p_ref[1]
    o_ref[...] = lax.dot_general(
        acc, w_ref[...], (((1,), (1,)), ((), ())),
        preferred_element_type=jnp.float32)


def _tc_matmul(partials, W_fc):
    return pl.pallas_call(
        _mm_body,
        grid=(N_NODES // ROWS_BLK,),
        in_specs=[
            pl.BlockSpec((NC, ROWS_BLK, CH), lambda i: (0, i, 0)),
            pl.BlockSpec((CH, CH), lambda i: (0, 0)),
        ],
        out_specs=pl.BlockSpec((ROWS_BLK, CH), lambda i: (i, 0)),
        out_shape=jax.ShapeDtypeStruct((N_NODES, CH), jnp.float32),
    )(partials, W_fc)


def kernel(x, edge_index, edge_attr, W_fc, W_edge, W_att):
    del edge_attr, W_edge, W_att
    ei = edge_index.astype(jnp.int32)
    pad = E_PAD - N_EDGES
    row1 = jnp.concatenate(
        [ei[0], jnp.full((pad,), N_NODES, jnp.int32),
         jnp.zeros((2 * K,), jnp.int32)])
    col1 = jnp.concatenate([ei[1], jnp.zeros((pad + 2 * K,), jnp.int32)])
    zeros = jnp.zeros((N_PAD, CH), jnp.float32)
    partials = _sc_aggregate(x, row1, col1, zeros)
    return _tc_matmul(partials[:, :N_NODES, :], W_fc)


# speedup vs baseline: 2.3273x; 1.7765x over previous
import functools

import jax
import jax.numpy as jnp
from jax import lax
from jax.experimental import pallas as pl
from jax.experimental.pallas import tpu as pltpu
from jax.experimental.pallas import tpu_sc as plsc

N_NODES = 10000
N_EDGES = 320000
CH = 128

NC = 2
NS = 16
NW = NC * NS
K = 80
CHUNKS = 125
EPW = CHUNKS * K
E_PAD = NW * EPW
NBUF = 2
N_PAD = 10240
RPT = N_PAD // NS


def _sc_aggregate(x, row1, col1, zeros):
    mesh = plsc.VectorSubcoreMesh(core_axis_name="c", subcore_axis_name="s")

    @functools.partial(
        pl.kernel,
        mesh=mesh,
        out_type=jax.ShapeDtypeStruct((NC, N_PAD, CH), jnp.float32),
        scratch_types=[
            pltpu.VMEM((NBUF, K), jnp.int32),
            pltpu.VMEM((NBUF, K), jnp.int32),
            pltpu.VMEM((K, CH), jnp.float32),
            pltpu.VMEM((K, CH), jnp.float32),
            pltpu.VMEM_SHARED((N_PAD, CH), jnp.float32),
            pltpu.SemaphoreType.DMA,
            pltpu.SemaphoreType.DMA,
            pltpu.SemaphoreType.DMA,
            pltpu.SemaphoreType.DMA,
        ],
    )
    def agg_kernel(x_hbm, row_hbm, col_hbm, z_hbm, out_hbm,
                   cbufs, rbufs, gbuf0, gbuf1, acc,
                   si0, si1, sg0, sg1):
        c = lax.axis_index("c")
        s = lax.axis_index("s")
        wid = c * NS + s
        base = wid * EPW

        gbuf = (gbuf0, gbuf1)
        sem_i = (si0, si1)
        sem_g = (sg0, sg1)

        def issue_idx(k, b):
            off = base + k * K
            pltpu.async_copy(col_hbm.at[pl.ds(off, K)], cbufs.at[b], sem_i[b])
            pltpu.async_copy(row_hbm.at[pl.ds(off, K)], rbufs.at[b], sem_i[b])

        def wait_idx(k, b):
            off = base + k * K
            pltpu.make_async_copy(col_hbm.at[pl.ds(off, K)], cbufs.at[b],
                                  sem_i[b]).wait()
            pltpu.make_async_copy(row_hbm.at[pl.ds(off, K)], rbufs.at[b],
                                  sem_i[b]).wait()

        def issue_gather(b):
            pltpu.async_copy(x_hbm.at[cbufs.at[b]], gbuf[b], sem_g[b])

        def wait_gather(b):
            pltpu.make_async_copy(x_hbm.at[cbufs.at[b]], gbuf[b],
                                  sem_g[b]).wait()

        issue_idx(0, 0)
        pltpu.sync_copy(z_hbm.at[pl.ds(s * RPT, RPT)],
                        acc.at[pl.ds(s * RPT, RPT)])
        wait_idx(0, 0)
        issue_gather(0)
        issue_idx(1, 1)
        plsc.subcore_barrier()

        def half(k, b):
            b2 = 1 - b
            wait_idx(k + 1, b2)
            issue_gather(b2)
            wait_gather(b)
            pltpu.sync_copy(gbuf[b], acc.at[rbufs.at[b]], add=True)
            issue_idx(k + 2, b)

        def body(g, carry):
            half(g * 2, 0)
            half(g * 2 + 1, 1)
            return carry

        lax.fori_loop(0, (CHUNKS - 1) // 2, body, 0)
        wait_gather(0)
        pltpu.sync_copy(gbuf[0], acc.at[rbufs.at[0]], add=True)
        wait_idx(CHUNKS, 1)

        plsc.subcore_barrier()
        pltpu.sync_copy(acc.at[pl.ds(s * RPT, RPT)],
                        out_hbm.at[c, pl.ds(s * RPT, RPT)])

    return agg_kernel(x, row1, col1, zeros)


ROWS_BLK = 2000


def _mm_body(p_ref, w_ref, o_ref):
    acc = p_ref[0] + p_ref[1]
    o_ref[...] = lax.dot_general(
        acc, w_ref[...], (((1,), (1,)), ((), ())),
        preferred_element_type=jnp.float32)


def _tc_matmul(partials, W_fc):
    return pl.pallas_call(
        _mm_body,
        grid=(N_NODES // ROWS_BLK,),
        in_specs=[
            pl.BlockSpec((NC, ROWS_BLK, CH), lambda i: (0, i, 0)),
            pl.BlockSpec((CH, CH), lambda i: (0, 0)),
        ],
        out_specs=pl.BlockSpec((ROWS_BLK, CH), lambda i: (i, 0)),
        out_shape=jax.ShapeDtypeStruct((N_NODES, CH), jnp.float32),
    )(partials, W_fc)


def kernel(x, edge_index, edge_attr, W_fc, W_edge, W_att):
    del edge_attr, W_edge, W_att
    ei = edge_index.astype(jnp.int32)
    pad = E_PAD - N_EDGES
    row1 = jnp.concatenate(
        [ei[0], jnp.full((pad,), N_NODES, jnp.int32),
         jnp.zeros((2 * K,), jnp.int32)])
    col1 = jnp.concatenate([ei[1], jnp.zeros((pad + 2 * K,), jnp.int32)])
    zeros = jnp.zeros((N_PAD, CH), jnp.float32)
    partials = _sc_aggregate(x, row1, col1, zeros)
    return _tc_matmul(partials[:, :N_NODES, :], W_fc)


# speedup vs baseline: 2.4216x; 1.0405x over previous
import functools

import jax
import jax.numpy as jnp
from jax import lax
from jax.experimental import pallas as pl
from jax.experimental.pallas import tpu as pltpu
from jax.experimental.pallas import tpu_sc as plsc

N_NODES = 10000
N_EDGES = 320000
CH = 128

NC = 2
NS = 16
NW = NC * NS
K = 80
CHUNKS = 125
EPW = CHUNKS * K
E_PAD = NW * EPW
NBUF = 2
N_PAD = 10240
RPT = N_PAD // NS


def _sc_aggregate(x, row1, col1, zeros):
    mesh = plsc.VectorSubcoreMesh(core_axis_name="c", subcore_axis_name="s")

    @functools.partial(
        pl.kernel,
        mesh=mesh,
        out_type=jax.ShapeDtypeStruct((NC, N_PAD, CH), jnp.float32),
        scratch_types=[
            pltpu.VMEM((NBUF, K), jnp.int32),
            pltpu.VMEM((NBUF, K), jnp.int32),
            pltpu.VMEM((K, CH), jnp.float32),
            pltpu.VMEM((K, CH), jnp.float32),
            pltpu.VMEM_SHARED((N_PAD, CH), jnp.float32),
            pltpu.SemaphoreType.DMA,
            pltpu.SemaphoreType.DMA,
            pltpu.SemaphoreType.DMA,
            pltpu.SemaphoreType.DMA,
        ],
    )
    def agg_kernel(x_hbm, row_hbm, col_hbm, z_hbm, out_hbm,
                   cbufs, rbufs, gbuf0, gbuf1, acc,
                   si0, si1, sg0, sg1):
        c = lax.axis_index("c")
        s = lax.axis_index("s")
        wid = c * NS + s
        base = wid * EPW

        gbuf = (gbuf0, gbuf1)
        sem_i = (si0, si1)
        sem_g = (sg0, sg1)

        def _off(k):
            return jnp.minimum(base + k * K, E_PAD - K)

        def issue_idx(k, b):
            off = _off(k)
            pltpu.async_copy(col_hbm.at[pl.ds(off, K)], cbufs.at[b], sem_i[b])
            pltpu.async_copy(row_hbm.at[pl.ds(off, K)], rbufs.at[b], sem_i[b])

        def wait_idx(k, b):
            off = _off(k)
            pltpu.make_async_copy(col_hbm.at[pl.ds(off, K)], cbufs.at[b],
                                  sem_i[b]).wait()
            pltpu.make_async_copy(row_hbm.at[pl.ds(off, K)], rbufs.at[b],
                                  sem_i[b]).wait()

        def issue_gather(b):
            pltpu.async_copy(x_hbm.at[cbufs.at[b]], gbuf[b], sem_g[b])

        def wait_gather(b):
            pltpu.make_async_copy(x_hbm.at[cbufs.at[b]], gbuf[b],
                                  sem_g[b]).wait()

        issue_idx(0, 0)
        pltpu.sync_copy(z_hbm.at[pl.ds(s * RPT, RPT)],
                        acc.at[pl.ds(s * RPT, RPT)])
        wait_idx(0, 0)
        issue_gather(0)
        issue_idx(1, 1)
        plsc.subcore_barrier()

        def half(k, b):
            b2 = 1 - b
            wait_idx(k + 1, b2)
            issue_gather(b2)
            wait_gather(b)
            pltpu.sync_copy(gbuf[b], acc.at[rbufs.at[b]], add=True)
            issue_idx(k + 2, b)

        def body(g, carry):
            half(g * 2, 0)
            half(g * 2 + 1, 1)
            return carry

        lax.fori_loop(0, (CHUNKS - 1) // 2, body, 0)
        wait_gather(0)
        pltpu.sync_copy(gbuf[0], acc.at[rbufs.at[0]], add=True)
        wait_idx(CHUNKS, 1)

        plsc.subcore_barrier()
        pltpu.sync_copy(acc.at[pl.ds(s * RPT, RPT)],
                        out_hbm.at[c, pl.ds(s * RPT, RPT)])

    return agg_kernel(x, row1, col1, zeros)


ROWS_BLK = 2000


def _mm_body(p_ref, w_ref, o_ref):
    acc = p_ref[0] + p_ref[1]
    o_ref[...] = lax.dot_general(
        acc, w_ref[...], (((1,), (1,)), ((), ())),
        preferred_element_type=jnp.float32)


def _tc_matmul(partials, W_fc):
    return pl.pallas_call(
        _mm_body,
        grid=(N_NODES // ROWS_BLK,),
        in_specs=[
            pl.BlockSpec((NC, ROWS_BLK, CH), lambda i: (0, i, 0)),
            pl.BlockSpec((CH, CH), lambda i: (0, 0)),
        ],
        out_specs=pl.BlockSpec((ROWS_BLK, CH), lambda i: (i, 0)),
        out_shape=jax.ShapeDtypeStruct((N_NODES, CH), jnp.float32),
    )(partials, W_fc)


def kernel(x, edge_index, edge_attr, W_fc, W_edge, W_att):
    del edge_attr, W_edge, W_att
    ei = edge_index.astype(jnp.int32)
    row1 = ei[0]
    col1 = ei[1]
    zeros = jnp.zeros((N_PAD, CH), jnp.float32)
    partials = _sc_aggregate(x, row1, col1, zeros)
    return _tc_matmul(partials, W_fc)


# speedup vs baseline: 2.5627x; 1.0583x over previous
import functools

import jax
import jax.numpy as jnp
from jax import lax
from jax.experimental import pallas as pl
from jax.experimental.pallas import tpu as pltpu
from jax.experimental.pallas import tpu_sc as plsc

N_NODES = 10000
N_EDGES = 320000
CH = 128

NC = 2
NS = 16
NW = NC * NS
K = 80
CHUNKS = 125
EPW = CHUNKS * K
E_PAD = NW * EPW
NBUF = 3
N_PAD = 10240
RPT = N_PAD // NS


def _sc_aggregate(x, row1, col1, zeros):
    mesh = plsc.VectorSubcoreMesh(core_axis_name="c", subcore_axis_name="s")

    @functools.partial(
        pl.kernel,
        mesh=mesh,
        out_type=jax.ShapeDtypeStruct((NC, N_PAD, CH), jnp.float32),
        scratch_types=[
            pltpu.VMEM((NBUF, K), jnp.int32),
            pltpu.VMEM((NBUF, K), jnp.int32),
            pltpu.VMEM((K, CH), jnp.float32),
            pltpu.VMEM((K, CH), jnp.float32),
            pltpu.VMEM((K, CH), jnp.float32),
            pltpu.VMEM_SHARED((N_PAD, CH), jnp.float32),
            pltpu.SemaphoreType.DMA,
            pltpu.SemaphoreType.DMA,
            pltpu.SemaphoreType.DMA,
            pltpu.SemaphoreType.DMA,
            pltpu.SemaphoreType.DMA,
            pltpu.SemaphoreType.DMA,
        ],
    )
    def agg_kernel(x_hbm, row_hbm, col_hbm, z_hbm, out_hbm,
                   cbufs, rbufs, gbuf0, gbuf1, gbuf2, acc,
                   si0, si1, si2, sg0, sg1, sg2):
        c = lax.axis_index("c")
        s = lax.axis_index("s")
        wid = c * NS + s
        base = wid * EPW

        gbuf = (gbuf0, gbuf1, gbuf2)
        sem_i = (si0, si1, si2)
        sem_g = (sg0, sg1, sg2)

        def _off(k):
            return jnp.minimum(base + k * K, E_PAD - K)

        def issue_idx(k, b):
            off = _off(k)
            pltpu.async_copy(col_hbm.at[pl.ds(off, K)], cbufs.at[b], sem_i[b])
            pltpu.async_copy(row_hbm.at[pl.ds(off, K)], rbufs.at[b], sem_i[b])

        def wait_idx(k, b):
            off = _off(k)
            pltpu.make_async_copy(col_hbm.at[pl.ds(off, K)], cbufs.at[b],
                                  sem_i[b]).wait()
            pltpu.make_async_copy(row_hbm.at[pl.ds(off, K)], rbufs.at[b],
                                  sem_i[b]).wait()

        def issue_gather(b):
            pltpu.async_copy(x_hbm.at[cbufs.at[b]], gbuf[b], sem_g[b])

        def wait_gather(b):
            pltpu.make_async_copy(x_hbm.at[cbufs.at[b]], gbuf[b],
                                  sem_g[b]).wait()

        issue_idx(0, 0)
        issue_idx(1, 1)
        pltpu.sync_copy(z_hbm.at[pl.ds(s * RPT, RPT)],
                        acc.at[pl.ds(s * RPT, RPT)])
        wait_idx(0, 0)
        issue_gather(0)
        wait_idx(1, 1)
        issue_gather(1)
        issue_idx(2, 2)
        plsc.subcore_barrier()

        def half(k, b):
            b2 = (b + 2) % NBUF
            wait_idx(k + 2, b2)
            issue_gather(b2)
            wait_gather(b)
            pltpu.sync_copy(gbuf[b], acc.at[rbufs.at[b]], add=True)
            issue_idx(k + 3, b)

        def body(g, carry):
            half(g * 3, 0)
            half(g * 3 + 1, 1)
            half(g * 3 + 2, 2)
            return carry

        lax.fori_loop(0, (CHUNKS - 2) // 3, body, 0)
        wait_gather((CHUNKS - 2) % NBUF)
        pltpu.sync_copy(gbuf[(CHUNKS - 2) % NBUF],
                        acc.at[rbufs.at[(CHUNKS - 2) % NBUF]], add=True)
        wait_gather((CHUNKS - 1) % NBUF)
        pltpu.sync_copy(gbuf[(CHUNKS - 1) % NBUF],
                        acc.at[rbufs.at[(CHUNKS - 1) % NBUF]], add=True)
        wait_idx(CHUNKS, CHUNKS % NBUF)

        plsc.subcore_barrier()
        pltpu.sync_copy(acc.at[pl.ds(s * RPT, RPT)],
                        out_hbm.at[c, pl.ds(s * RPT, RPT)])

    return agg_kernel(x, row1, col1, zeros)


ROWS_BLK = 2000


def _mm_body(p_ref, w_ref, o_ref):
    acc = p_ref[0] + p_ref[1]
    o_ref[...] = lax.dot_general(
        acc, w_ref[...], (((1,), (1,)), ((), ())),
        preferred_element_type=jnp.float32)


def _tc_matmul(partials, W_fc):
    return pl.pallas_call(
        _mm_body,
        grid=(N_NODES // ROWS_BLK,),
        in_specs=[
            pl.BlockSpec((NC, ROWS_BLK, CH), lambda i: (0, i, 0)),
            pl.BlockSpec((CH, CH), lambda i: (0, 0)),
        ],
        out_specs=pl.BlockSpec((ROWS_BLK, CH), lambda i: (i, 0)),
        out_shape=jax.ShapeDtypeStruct((N_NODES, CH), jnp.float32),
    )(partials, W_fc)


def kernel(x, edge_index, edge_attr, W_fc, W_edge, W_att):
    del edge_attr, W_edge, W_att
    ei = edge_index.astype(jnp.int32)
    row1 = ei[0]
    col1 = ei[1]
    zeros = jnp.zeros((N_PAD, CH), jnp.float32)
    partials = _sc_aggregate(x, row1, col1, zeros)
    return _tc_matmul(partials, W_fc)


# speedup vs baseline: 2.5707x; 1.0031x over previous
import functools

import jax
import jax.numpy as jnp
from jax import lax
from jax.experimental import pallas as pl
from jax.experimental.pallas import tpu as pltpu
from jax.experimental.pallas import tpu_sc as plsc

N_NODES = 10000
N_EDGES = 320000
CH = 128

NC = 2
NS = 16
NW = NC * NS
K = 80
CHUNKS = 125
EPW = CHUNKS * K
E_PAD = NW * EPW
NBUF = 4
N_PAD = 10240
RPT = N_PAD // NS


def _sc_aggregate(x, row1, col1, zeros):
    mesh = plsc.VectorSubcoreMesh(core_axis_name="c", subcore_axis_name="s")

    @functools.partial(
        pl.kernel,
        mesh=mesh,
        out_type=jax.ShapeDtypeStruct((NC, N_PAD, CH), jnp.float32),
        scratch_types=[
            pltpu.VMEM((NBUF, K), jnp.int32),
            pltpu.VMEM((NBUF, K), jnp.int32),
            pltpu.VMEM((K, CH), jnp.float32),
            pltpu.VMEM((K, CH), jnp.float32),
            pltpu.VMEM((K, CH), jnp.float32),
            pltpu.VMEM((K, CH), jnp.float32),
            pltpu.VMEM_SHARED((N_PAD, CH), jnp.float32),
            pltpu.SemaphoreType.DMA,
            pltpu.SemaphoreType.DMA,
            pltpu.SemaphoreType.DMA,
            pltpu.SemaphoreType.DMA,
            pltpu.SemaphoreType.DMA,
            pltpu.SemaphoreType.DMA,
            pltpu.SemaphoreType.DMA,
            pltpu.SemaphoreType.DMA,
        ],
    )
    def agg_kernel(x_hbm, row_hbm, col_hbm, z_hbm, out_hbm,
                   cbufs, rbufs, gbuf0, gbuf1, gbuf2, gbuf3, acc,
                   si0, si1, si2, si3, sg0, sg1, sg2, sg3):
        c = lax.axis_index("c")
        s = lax.axis_index("s")
        wid = c * NS + s
        base = wid * EPW

        gbuf = (gbuf0, gbuf1, gbuf2, gbuf3)
        sem_i = (si0, si1, si2, si3)
        sem_g = (sg0, sg1, sg2, sg3)

        def _off(k):
            return jnp.minimum(base + k * K, E_PAD - K)

        def issue_idx(k, b):
            off = _off(k)
            pltpu.async_copy(col_hbm.at[pl.ds(off, K)], cbufs.at[b], sem_i[b])
            pltpu.async_copy(row_hbm.at[pl.ds(off, K)], rbufs.at[b], sem_i[b])

        def wait_idx(k, b):
            off = _off(k)
            pltpu.make_async_copy(col_hbm.at[pl.ds(off, K)], cbufs.at[b],
                                  sem_i[b]).wait()
            pltpu.make_async_copy(row_hbm.at[pl.ds(off, K)], rbufs.at[b],
                                  sem_i[b]).wait()

        def issue_gather(b):
            pltpu.async_copy(x_hbm.at[cbufs.at[b]], gbuf[b], sem_g[b])

        def wait_gather(b):
            pltpu.make_async_copy(x_hbm.at[cbufs.at[b]], gbuf[b],
                                  sem_g[b]).wait()

        issue_idx(0, 0)
        issue_idx(1, 1)
        pltpu.sync_copy(z_hbm.at[pl.ds(s * RPT, RPT)],
                        acc.at[pl.ds(s * RPT, RPT)])
        wait_idx(0, 0)
        issue_gather(0)
        wait_idx(1, 1)
        issue_gather(1)
        issue_idx(2, 2)
        issue_idx(3, 3)
        wait_idx(2, 2)
        issue_gather(2)
        plsc.subcore_barrier()

        def scatter(b):
            pltpu.sync_copy(gbuf[b], acc.at[rbufs.at[b]], add=True)

        def half(k, b):
            b2 = (b + 3) % NBUF
            wait_idx(k + 3, b2)
            issue_gather(b2)
            wait_gather(b)
            scatter(b)
            issue_idx(k + 4, b)

        def body(g, carry):
            half(g * 4, 0)
            half(g * 4 + 1, 1)
            half(g * 4 + 2, 2)
            half(g * 4 + 3, 3)
            return carry

        lax.fori_loop(0, (CHUNKS - 5) // 4, body, 0)
        half(CHUNKS - 5, (CHUNKS - 5) % NBUF)
        half(CHUNKS - 4, (CHUNKS - 4) % NBUF)
        wait_gather((CHUNKS - 3) % NBUF)
        scatter((CHUNKS - 3) % NBUF)
        wait_gather((CHUNKS - 2) % NBUF)
        scatter((CHUNKS - 2) % NBUF)
        wait_gather((CHUNKS - 1) % NBUF)
        scatter((CHUNKS - 1) % NBUF)
        wait_idx(CHUNKS, CHUNKS % NBUF)

        plsc.subcore_barrier()
        pltpu.sync_copy(acc.at[pl.ds(s * RPT, RPT)],
                        out_hbm.at[c, pl.ds(s * RPT, RPT)])

    return agg_kernel(x, row1, col1, zeros)


ROWS_BLK = 2000


def _mm_body(p_ref, w_ref, o_ref):
    acc = p_ref[0] + p_ref[1]
    o_ref[...] = lax.dot_general(
        acc, w_ref[...], (((1,), (1,)), ((), ())),
        preferred_element_type=jnp.float32)


def _tc_matmul(partials, W_fc):
    return pl.pallas_call(
        _mm_body,
        grid=(N_NODES // ROWS_BLK,),
        in_specs=[
            pl.BlockSpec((NC, ROWS_BLK, CH), lambda i: (0, i, 0)),
            pl.BlockSpec((CH, CH), lambda i: (0, 0)),
        ],
        out_specs=pl.BlockSpec((ROWS_BLK, CH), lambda i: (i, 0)),
        out_shape=jax.ShapeDtypeStruct((N_NODES, CH), jnp.float32),
    )(partials, W_fc)


def kernel(x, edge_index, edge_attr, W_fc, W_edge, W_att):
    del edge_attr, W_edge, W_att
    ei = edge_index.astype(jnp.int32)
    row1 = ei[0]
    col1 = ei[1]
    zeros = jnp.zeros((N_PAD, CH), jnp.float32)
    partials = _sc_aggregate(x, row1, col1, zeros)
    return _tc_matmul(partials, W_fc)


# speedup vs baseline: 2.6428x; 1.0280x over previous
import functools

import jax
import jax.numpy as jnp
from jax import lax
from jax.experimental import pallas as pl
from jax.experimental.pallas import tpu as pltpu
from jax.experimental.pallas import tpu_sc as plsc

N_NODES = 10000
N_EDGES = 320000
CH = 128

NC = 2
NS = 16
NW = NC * NS
K = 80
CHUNKS = 125
EPW = CHUNKS * K
E_PAD = NW * EPW
NBUF = 4
N_PAD = 10240
RPT = N_PAD // NS


def _sc_aggregate(x, row1, col1):
    mesh = plsc.VectorSubcoreMesh(core_axis_name="c", subcore_axis_name="s")

    @functools.partial(
        pl.kernel,
        mesh=mesh,
        out_type=jax.ShapeDtypeStruct((NC, N_PAD, CH), jnp.float32),
        scratch_types=[
            pltpu.VMEM((NBUF, K), jnp.int32),
            pltpu.VMEM((NBUF, K), jnp.int32),
            pltpu.VMEM((K, CH), jnp.float32),
            pltpu.VMEM((K, CH), jnp.float32),
            pltpu.VMEM((K, CH), jnp.float32),
            pltpu.VMEM((K, CH), jnp.float32),
            pltpu.VMEM_SHARED((N_PAD, CH), jnp.float32),
            pltpu.SemaphoreType.DMA,
            pltpu.SemaphoreType.DMA,
            pltpu.SemaphoreType.DMA,
            pltpu.SemaphoreType.DMA,
            pltpu.SemaphoreType.DMA,
            pltpu.SemaphoreType.DMA,
            pltpu.SemaphoreType.DMA,
            pltpu.SemaphoreType.DMA,
        ],
    )
    def agg_kernel(x_hbm, row_hbm, col_hbm, out_hbm,
                   cbufs, rbufs, gbuf0, gbuf1, gbuf2, gbuf3, acc,
                   si0, si1, si2, si3, sg0, sg1, sg2, sg3):
        c = lax.axis_index("c")
        s = lax.axis_index("s")
        wid = c * NS + s
        base = wid * EPW

        gbuf = (gbuf0, gbuf1, gbuf2, gbuf3)
        sem_i = (si0, si1, si2, si3)
        sem_g = (sg0, sg1, sg2, sg3)

        def _off(k):
            return jnp.minimum(base + k * K, E_PAD - K)

        def issue_idx(k, b):
            off = _off(k)
            pltpu.async_copy(col_hbm.at[pl.ds(off, K)], cbufs.at[b], sem_i[b])
            pltpu.async_copy(row_hbm.at[pl.ds(off, K)], rbufs.at[b], sem_i[b])

        def wait_idx(k, b):
            off = _off(k)
            pltpu.make_async_copy(col_hbm.at[pl.ds(off, K)], cbufs.at[b],
                                  sem_i[b]).wait()
            pltpu.make_async_copy(row_hbm.at[pl.ds(off, K)], rbufs.at[b],
                                  sem_i[b]).wait()

        def issue_gather(b):
            pltpu.async_copy(x_hbm.at[cbufs.at[b]], gbuf[b], sem_g[b])

        def wait_gather(b):
            pltpu.make_async_copy(x_hbm.at[cbufs.at[b]], gbuf[b],
                                  sem_g[b]).wait()

        issue_idx(0, 0)
        issue_idx(1, 1)
        zv = jnp.zeros((16,), jnp.float32)

        def zrow(i, carry):
            for j in range(CH // 16):
                gbuf0[i, pl.ds(j * 16, 16)] = zv
            return carry

        lax.fori_loop(0, K, zrow, 0)
        for t in range(RPT // K):
            pltpu.sync_copy(gbuf0, acc.at[pl.ds(s * RPT + t * K, K)])
        wait_idx(0, 0)
        issue_gather(0)
        wait_idx(1, 1)
        issue_gather(1)
        issue_idx(2, 2)
        issue_idx(3, 3)
        wait_idx(2, 2)
        issue_gather(2)
        plsc.subcore_barrier()

        def scatter(b):
            pltpu.sync_copy(gbuf[b], acc.at[rbufs.at[b]], add=True)

        def half(k, b):
            b2 = (b + 3) % NBUF
            wait_idx(k + 3, b2)
            issue_gather(b2)
            wait_gather(b)
            scatter(b)
            issue_idx(k + 4, b)

        def body(g, carry):
            half(g * 4, 0)
            half(g * 4 + 1, 1)
            half(g * 4 + 2, 2)
            half(g * 4 + 3, 3)
            return carry

        lax.fori_loop(0, (CHUNKS - 5) // 4, body, 0)
        half(CHUNKS - 5, (CHUNKS - 5) % NBUF)
        half(CHUNKS - 4, (CHUNKS - 4) % NBUF)
        wait_gather((CHUNKS - 3) % NBUF)
        scatter((CHUNKS - 3) % NBUF)
        wait_gather((CHUNKS - 2) % NBUF)
        scatter((CHUNKS - 2) % NBUF)
        wait_gather((CHUNKS - 1) % NBUF)
        scatter((CHUNKS - 1) % NBUF)
        wait_idx(CHUNKS, CHUNKS % NBUF)

        plsc.subcore_barrier()
        pltpu.sync_copy(acc.at[pl.ds(s * RPT, RPT)],
                        out_hbm.at[c, pl.ds(s * RPT, RPT)])

    return agg_kernel(x, row1, col1)


ROWS_BLK = 2000


def _mm_body(p_ref, w_ref, o_ref):
    acc = p_ref[0] + p_ref[1]
    o_ref[...] = lax.dot_general(
        acc, w_ref[...], (((1,), (1,)), ((), ())),
        preferred_element_type=jnp.float32)


def _tc_matmul(partials, W_fc):
    return pl.pallas_call(
        _mm_body,
        grid=(N_NODES // ROWS_BLK,),
        in_specs=[
            pl.BlockSpec((NC, ROWS_BLK, CH), lambda i: (0, i, 0)),
            pl.BlockSpec((CH, CH), lambda i: (0, 0)),
        ],
        out_specs=pl.BlockSpec((ROWS_BLK, CH), lambda i: (i, 0)),
        out_shape=jax.ShapeDtypeStruct((N_NODES, CH), jnp.float32),
    )(partials, W_fc)


def kernel(x, edge_index, edge_attr, W_fc, W_edge, W_att):
    del edge_attr, W_edge, W_att
    ei = edge_index.astype(jnp.int32)
    row1 = ei[0]
    col1 = ei[1]
    partials = _sc_aggregate(x, row1, col1)
    return _tc_matmul(partials, W_fc)


# speedup vs baseline: 2.8154x; 1.0653x over previous
import functools

import jax
import jax.numpy as jnp
from jax import lax
from jax.experimental import pallas as pl
from jax.experimental.pallas import tpu as pltpu
from jax.experimental.pallas import tpu_sc as plsc

N_NODES = 10000
N_EDGES = 320000
CH = 128

NC = 2
NS = 16
NW = NC * NS
K = 80
CHUNKS = 125
EPW = CHUNKS * K
E_PAD = NW * EPW
NBUF = 4
N_PAD = 10240
RPT = N_PAD // NS


def _sc_aggregate(x, eflat):
    mesh = plsc.VectorSubcoreMesh(core_axis_name="c", subcore_axis_name="s")

    @functools.partial(
        pl.kernel,
        mesh=mesh,
        out_type=jax.ShapeDtypeStruct((NC, N_PAD, CH), jnp.float32),
        scratch_types=[
            pltpu.VMEM((NBUF, K), jnp.int32),
            pltpu.VMEM((NBUF, K), jnp.int32),
            pltpu.VMEM((K, CH), jnp.float32),
            pltpu.VMEM((K, CH), jnp.float32),
            pltpu.VMEM((K, CH), jnp.float32),
            pltpu.VMEM((K, CH), jnp.float32),
            pltpu.VMEM_SHARED((N_PAD, CH), jnp.float32),
            pltpu.SemaphoreType.DMA,
            pltpu.SemaphoreType.DMA,
            pltpu.SemaphoreType.DMA,
            pltpu.SemaphoreType.DMA,
            pltpu.SemaphoreType.DMA,
            pltpu.SemaphoreType.DMA,
            pltpu.SemaphoreType.DMA,
            pltpu.SemaphoreType.DMA,
        ],
    )
    def agg_kernel(x_hbm, e_hbm, out_hbm,
                   cbufs, rbufs, gbuf0, gbuf1, gbuf2, gbuf3, acc,
                   si0, si1, si2, si3, sg0, sg1, sg2, sg3):
        c = lax.axis_index("c")
        s = lax.axis_index("s")
        wid = c * NS + s
        base = wid * EPW

        gbuf = (gbuf0, gbuf1, gbuf2, gbuf3)
        sem_i = (si0, si1, si2, si3)
        sem_g = (sg0, sg1, sg2, sg3)

        def _off(k):
            return jnp.minimum(base + k * K, E_PAD - K)

        def issue_idx(k, b):
            off = _off(k)
            pltpu.async_copy(e_hbm.at[pl.ds(E_PAD + off, K)], cbufs.at[b],
                             sem_i[b])
            pltpu.async_copy(e_hbm.at[pl.ds(off, K)], rbufs.at[b], sem_i[b])

        def wait_idx(k, b):
            off = _off(k)
            pltpu.make_async_copy(e_hbm.at[pl.ds(E_PAD + off, K)],
                                  cbufs.at[b], sem_i[b]).wait()
            pltpu.make_async_copy(e_hbm.at[pl.ds(off, K)], rbufs.at[b],
                                  sem_i[b]).wait()

        def issue_gather(b):
            pltpu.async_copy(x_hbm.at[cbufs.at[b]], gbuf[b], sem_g[b])

        def wait_gather(b):
            pltpu.make_async_copy(x_hbm.at[cbufs.at[b]], gbuf[b],
                                  sem_g[b]).wait()

        issue_idx(0, 0)
        issue_idx(1, 1)
        zv = jnp.zeros((16,), jnp.float32)

        def zrow(i, carry):
            for j in range(CH // 16):
                gbuf0[i, pl.ds(j * 16, 16)] = zv
            return carry

        lax.fori_loop(0, K, zrow, 0)
        for t in range(RPT // K):
            pltpu.sync_copy(gbuf0, acc.at[pl.ds(s * RPT + t * K, K)])
        wait_idx(0, 0)
        issue_gather(0)
        wait_idx(1, 1)
        issue_gather(1)
        issue_idx(2, 2)
        issue_idx(3, 3)
        wait_idx(2, 2)
        issue_gather(2)
        plsc.subcore_barrier()

        def scatter(b):
            pltpu.sync_copy(gbuf[b], acc.at[rbufs.at[b]], add=True)

        def half(k, b):
            b2 = (b + 3) % NBUF
            wait_idx(k + 3, b2)
            issue_gather(b2)
            wait_gather(b)
            scatter(b)
            issue_idx(k + 4, b)

        def body(g, carry):
            half(g * 4, 0)
            half(g * 4 + 1, 1)
            half(g * 4 + 2, 2)
            half(g * 4 + 3, 3)
            return carry

        lax.fori_loop(0, (CHUNKS - 5) // 4, body, 0)
        half(CHUNKS - 5, (CHUNKS - 5) % NBUF)
        half(CHUNKS - 4, (CHUNKS - 4) % NBUF)
        wait_gather((CHUNKS - 3) % NBUF)
        scatter((CHUNKS - 3) % NBUF)
        wait_gather((CHUNKS - 2) % NBUF)
        scatter((CHUNKS - 2) % NBUF)
        wait_gather((CHUNKS - 1) % NBUF)
        scatter((CHUNKS - 1) % NBUF)
        wait_idx(CHUNKS, CHUNKS % NBUF)

        plsc.subcore_barrier()
        pltpu.sync_copy(acc.at[pl.ds(s * RPT, RPT)],
                        out_hbm.at[c, pl.ds(s * RPT, RPT)])

    return agg_kernel(x, eflat)


ROWS_BLK = 2000


def _mm_body(p_ref, w_ref, o_ref):
    acc = p_ref[0] + p_ref[1]
    o_ref[...] = lax.dot_general(
        acc, w_ref[...], (((1,), (1,)), ((), ())),
        preferred_element_type=jnp.float32)


def _tc_matmul(partials, W_fc):
    return pl.pallas_call(
        _mm_body,
        grid=(N_NODES // ROWS_BLK,),
        in_specs=[
            pl.BlockSpec((NC, ROWS_BLK, CH), lambda i: (0, i, 0)),
            pl.BlockSpec((CH, CH), lambda i: (0, 0)),
        ],
        out_specs=pl.BlockSpec((ROWS_BLK, CH), lambda i: (i, 0)),
        out_shape=jax.ShapeDtypeStruct((N_NODES, CH), jnp.float32),
    )(partials, W_fc)


def kernel(x, edge_index, edge_attr, W_fc, W_edge, W_att):
    del edge_attr, W_edge, W_att
    eflat = edge_index.astype(jnp.int32).reshape(-1)
    partials = _sc_aggregate(x, eflat)
    return _tc_matmul(partials, W_fc)
